# Initial kernel scaffold; baseline (speedup 1.0000x reference)
import jax
import jax.numpy as jnp
from jax.experimental import pallas as pl


def kernel(x):
    raise NotImplementedError("write your pallas kernel here")


# speedup vs baseline: 1.6798x; 1.6798x over previous
import functools

import jax
import jax.numpy as jnp
from jax.experimental import pallas as pl
from jax.experimental.pallas import tpu as pltpu


def _body(a_ref, b2_ref, out_ref, stash_ref, mn_ref, mx_ref, *, nb):
    t = pl.program_id(1)

    @pl.when(t < nb)
    def _load():
        a = a_ref[0, 0, 0]
        s = jnp.where(t == 0, b2_ref[0, 0, 0] - a, a)
        stash_ref[t] = s
        m = jnp.min(s, axis=0, keepdims=True)
        mm = jnp.max(s, axis=0, keepdims=True)

        @pl.when(t == 0)
        def _():
            mn_ref[...] = m
            mx_ref[...] = mm

        @pl.when(t != 0)
        def _():
            mn_ref[...] = jnp.minimum(mn_ref[...], m)
            mx_ref[...] = jnp.maximum(mx_ref[...], mm)

    @pl.when(t >= nb)
    def _store():
        s = stash_ref[t - nb]
        mn = mn_ref[...]
        rng = mx_ref[...] - mn
        denom = jnp.where(rng == 0.0, 1.0, rng)
        out_ref[0, 0] = (s - mn) / denom


def kernel(x):
    bs, c, n, h, w = x.shape
    nb = bs * c
    cores = 2
    wc = w // cores

    body = functools.partial(_body, nb=nb)
    out = pl.pallas_call(
        body,
        grid=(cores, 2 * nb),
        in_specs=[
            pl.BlockSpec(
                (1, 1, 1, h, wc),
                lambda i, t: (jnp.minimum(t, nb - 1), 0, 0, 0, i),
            ),
            pl.BlockSpec(
                (1, 1, 1, h, wc),
                lambda i, t: (0, 0, 2, 0, i),
            ),
        ],
        out_specs=pl.BlockSpec(
            (1, 1, h, wc),
            lambda i, t: (jnp.maximum(t - nb, 0), 0, 0, i),
        ),
        out_shape=jax.ShapeDtypeStruct((nb, 1, h, w), x.dtype),
        scratch_shapes=[
            pltpu.VMEM((nb, h, wc), jnp.float32),
            pltpu.VMEM((1, wc), jnp.float32),
            pltpu.VMEM((1, wc), jnp.float32),
        ],
        compiler_params=pltpu.CompilerParams(
            dimension_semantics=("parallel", "arbitrary"),
            vmem_limit_bytes=56 * 1024 * 1024,
        ),
    )(x, x)
    return out.reshape(bs, c, h, w)


# speedup vs baseline: 2.1252x; 1.2652x over previous
import functools

import jax
import jax.numpy as jnp
from jax.experimental import pallas as pl
from jax.experimental.pallas import tpu as pltpu

_PB = 2


def _body(a_ref, b2_ref, out_ref, stash_ref, mn_ref, mx_ref, *, nsteps, h, wc):
    t = pl.program_id(1)

    @pl.when(t == 0)
    def _load_first():
        a = a_ref[:, 0, 0]
        s0 = b2_ref[0, 0, 0] - a[0]
        s1 = a[1]
        stash_ref[0] = s0
        stash_ref[1] = s1
        mn_ref[...] = jnp.minimum(
            jnp.min(s0, axis=0, keepdims=True), jnp.min(s1, axis=0, keepdims=True)
        )
        mx_ref[...] = jnp.maximum(
            jnp.max(s0, axis=0, keepdims=True), jnp.max(s1, axis=0, keepdims=True)
        )

    @pl.when(jnp.logical_and(t > 0, t < nsteps))
    def _load():
        a = a_ref[:, 0, 0].reshape(_PB * h, wc)
        stash_ref[pl.ds(t * _PB, _PB)] = a.reshape(_PB, h, wc)
        mn_ref[...] = jnp.minimum(mn_ref[...], jnp.min(a, axis=0, keepdims=True))
        mx_ref[...] = jnp.maximum(mx_ref[...], jnp.max(a, axis=0, keepdims=True))

    @pl.when(t >= nsteps)
    def _store():
        s = stash_ref[pl.ds((t - nsteps) * _PB, _PB)]
        mn = mn_ref[...]
        rng = mx_ref[...] - mn
        inv = 1.0 / jnp.where(rng == 0.0, 1.0, rng)
        out_ref[:, 0] = (s - mn) * inv


def kernel(x):
    bs, c, n, h, w = x.shape
    nb = bs * c
    cores = 2
    wc = w // cores
    nsteps = nb // _PB

    body = functools.partial(_body, nsteps=nsteps, h=h, wc=wc)
    out = pl.pallas_call(
        body,
        grid=(cores, 2 * nsteps),
        in_specs=[
            pl.BlockSpec(
                (_PB, 1, 1, h, wc),
                lambda i, t: (jnp.minimum(t, nsteps - 1), 0, 0, 0, i),
            ),
            pl.BlockSpec(
                (1, 1, 1, h, wc),
                lambda i, t: (0, 0, 2, 0, i),
            ),
        ],
        out_specs=pl.BlockSpec(
            (_PB, 1, h, wc),
            lambda i, t: (jnp.maximum(t - nsteps, 0), 0, 0, i),
        ),
        out_shape=jax.ShapeDtypeStruct((nb, 1, h, w), x.dtype),
        scratch_shapes=[
            pltpu.VMEM((nb, h, wc), jnp.float32),
            pltpu.VMEM((1, wc), jnp.float32),
            pltpu.VMEM((1, wc), jnp.float32),
        ],
        compiler_params=pltpu.CompilerParams(
            dimension_semantics=("parallel", "arbitrary"),
            vmem_limit_bytes=56 * 1024 * 1024,
        ),
    )(x, x)
    return out.reshape(bs, c, h, w)
